# baseline (speedup 1.0000x reference)
import functools

import jax
import jax.numpy as jnp
from jax import lax
from jax.experimental import pallas as pl
from jax.experimental.pallas import tpu as pltpu
from jax.experimental.pallas import tpu_sc as plsc

L = 200
B = 1024
E = 64
V = 1000000
LANES = 16
NBG = 8
BGW = B // NBG
NLG = 4
PH = L // NLG
NG = BGW // LANES
FW = 2 * E + 1
SROWS = 56


@functools.partial(
    pl.kernel,
    out_type=jax.ShapeDtypeStruct((L, E, B), jnp.float32),
    mesh=plsc.VectorSubcoreMesh(core_axis_name="c", subcore_axis_name="s"),
    scratch_types=[
        pltpu.VMEM((SROWS, BGW), jnp.int32),
        pltpu.VMEM((E, L), jnp.float32),
        pltpu.VMEM((BGW, FW), jnp.float32),
        pltpu.VMEM((BGW, FW), jnp.float32),
        pltpu.VMEM((BGW, FW), jnp.float32),
        pltpu.VMEM((E, BGW), jnp.float32),
        pltpu.SemaphoreType.DMA,
        pltpu.SemaphoreType.DMA,
        pltpu.SemaphoreType.DMA,
        pltpu.SemaphoreType.DMA,
    ],
    compiler_params=pltpu.CompilerParams(needs_layout_passes=False),
)
def _embed_kernel(idx_hbm, table_hbm, post_hbm, out_hbm,
                  idx_v, post_v, frows0, frows1, frows2, trows_v,
                  gsem0, gsem1, gsem2, osem):
    wid = lax.axis_index("s") * 2 + lax.axis_index("c")
    bg = wid % NBG
    lg = wid // NBG
    b0 = bg * BGW
    l0 = lg * PH
    l0a = (l0 // 8) * 8
    off0 = l0 - l0a

    frows = (frows0, frows1, frows2)
    gsem = (gsem0, gsem1, gsem2)

    pltpu.sync_copy(idx_hbm.at[pl.ds(l0a, SROWS), pl.ds(b0, BGW)], idx_v)
    pltpu.sync_copy(post_hbm, post_v)

    iota16 = lax.iota(jnp.int32, LANES)

    def g_copy(u, j):
        return pltpu.make_async_copy(
            table_hbm.at[idx_v.at[off0 + u]], frows[j].at[:, pl.ds(0, 2 * E)],
            gsem[j])

    def o_copy(u):
        l = l0 + u
        return pltpu.make_async_copy(
            trows_v, out_hbm.at[l, :, pl.ds(b0, BGW)], osem)

    def process(u, j):
        l = l0 + u
        rids = [iota16 + g * LANES for g in range(NG)]
        lsplat = jnp.full((LANES,), l, jnp.int32)

        @plsc.parallel_loop(0, E, unroll=2)
        def ebody(e):
            esplat = jnp.full((LANES,), e, jnp.int32)
            pv = plsc.load_gather(post_v, [esplat, lsplat])
            for g in range(NG):
                val = plsc.load_gather(frows[j], [rids[g], esplat])
                trows_v[e, pl.ds(g * LANES, LANES)] = val + pv

    def unit(u, j):
        @pl.when(u + 2 < PH)
        def _():
            g_copy(u + 2, (j + 2) % 3).start()

        g_copy(u, j).wait()

        @pl.when(u >= 1)
        def _():
            o_copy(u - 1).wait()

        process(u, j)
        o_copy(u).start()

    g_copy(0, 0).start()
    g_copy(1, 1).start()

    def body(i, carry):
        u0 = 3 * i
        for k in range(3):
            unit(u0 + k, k)
        return carry

    lax.fori_loop(0, PH // 3, body, 0)
    for k in range(48, PH):
        unit(k, k % 3)
    o_copy(PH - 1).wait()


def kernel(inputs, embedding, posembedding):
    table = jnp.pad(embedding, ((0, 0), (0, E)))
    post = posembedding.T
    outt = _embed_kernel(inputs.astype(jnp.int32), table, post)
    return jnp.transpose(outt, (0, 2, 1))


# speedup vs baseline: 1.0192x; 1.0192x over previous
import functools

import jax
import jax.numpy as jnp
from jax import lax
from jax.experimental import pallas as pl
from jax.experimental.pallas import tpu as pltpu
from jax.experimental.pallas import tpu_sc as plsc

L = 200
B = 1024
E = 64
V = 1000000
LANES = 16
NBG = 8
BGW = B // NBG
NLG = 4
PH = L // NLG
NG = BGW // LANES
FW = 2 * E + 1


@functools.partial(
    pl.kernel,
    out_type=jax.ShapeDtypeStruct((L, E, B), jnp.float32),
    mesh=plsc.VectorSubcoreMesh(core_axis_name="c", subcore_axis_name="s"),
    scratch_types=[
        pltpu.VMEM((L, BGW), jnp.int32),
        pltpu.VMEM((E, L), jnp.float32),
        pltpu.VMEM((BGW, FW), jnp.float32),
        pltpu.VMEM((BGW, FW), jnp.float32),
        pltpu.VMEM((E, BGW), jnp.float32),
        pltpu.VMEM((E, BGW), jnp.float32),
        pltpu.SemaphoreType.DMA,
        pltpu.SemaphoreType.DMA,
        pltpu.SemaphoreType.DMA,
        pltpu.SemaphoreType.DMA,
    ],
    compiler_params=pltpu.CompilerParams(needs_layout_passes=False),
)
def _embed_kernel(idx_hbm, table_hbm, post_hbm, out_hbm,
                  idx_v, post_v, frows0, frows1, trows0, trows1,
                  gsem0, gsem1, osem0, osem1):
    wid = lax.axis_index("s") * 2 + lax.axis_index("c")
    bg = wid % NBG
    lg = wid // NBG
    b0 = bg * BGW
    l0 = lg * PH

    frows = (frows0, frows1)
    trows = (trows0, trows1)
    gsem = (gsem0, gsem1)
    osem = (osem0, osem1)

    pltpu.sync_copy(idx_hbm.at[:, pl.ds(b0, BGW)], idx_v)
    pltpu.sync_copy(post_hbm, post_v)

    iota16 = lax.iota(jnp.int32, LANES)

    def g_copy(u, j):
        return pltpu.make_async_copy(
            table_hbm.at[idx_v.at[l0 + u]], frows[j].at[:, pl.ds(0, 2 * E)],
            gsem[j])

    def o_copy(u, j):
        l = l0 + u
        return pltpu.make_async_copy(
            trows[j], out_hbm.at[l, :, pl.ds(b0, BGW)], osem[j])

    def process(u, j):
        l = l0 + u
        rids = [iota16 + g * LANES for g in range(NG)]
        lsplat = jnp.full((LANES,), l, jnp.int32)

        @plsc.parallel_loop(0, E, unroll=2)
        def ebody(e):
            esplat = jnp.full((LANES,), e, jnp.int32)
            pv = plsc.load_gather(post_v, [esplat, lsplat])
            for g in range(NG):
                val = plsc.load_gather(frows[j], [rids[g], esplat])
                trows[j][e, pl.ds(g * LANES, LANES)] = val + pv

    def unit(u, j):
        @pl.when(u + 1 < PH)
        def _():
            g_copy(u + 1, 1 - j).start()

        g_copy(u, j).wait()

        @pl.when(u >= 2)
        def _():
            o_copy(u - 2, j).wait()

        process(u, j)
        o_copy(u, j).start()

    g_copy(0, 0).start()

    def body(i, carry):
        unit(2 * i, 0)
        unit(2 * i + 1, 1)
        return carry

    lax.fori_loop(0, PH // 2, body, 0)
    o_copy(PH - 2, 0).wait()
    o_copy(PH - 1, 1).wait()


def kernel(inputs, embedding, posembedding):
    table = jnp.pad(embedding, ((0, 0), (0, E)))
    post = posembedding.T
    outt = _embed_kernel(inputs.astype(jnp.int32), table, post)
    return jnp.transpose(outt, (0, 2, 1))
